# baseline (speedup 1.0000x reference)
import dataclasses
import functools

import jax
import jax.numpy as jnp
from jax.experimental import pallas as pl
from jax.experimental.pallas import tpu as pltpu
from jax.experimental.pallas import tpu_sc as plsc

N = 10000
E = 160000
D = 256
H = D // 2
NC = 2
NS = 16
L = 16
CHUNK = 64
NCHUNKS = E // CHUNK
RCHUNK = 128
NRCH = N // RCHUNK
RTAIL = N - NRCH * RCHUNK

_mesh = plsc.VectorSubcoreMesh(
    core_axis_name="c", subcore_axis_name="s", num_cores=NC, num_subcores=NS
)

_sc_params = pltpu.CompilerParams()
if "needs_layout_passes" in pltpu.CompilerParams.__dataclass_fields__:
    _sc_params = dataclasses.replace(_sc_params, needs_layout_passes=False)


def _prep_body(x_ref, xa_ref, xb_ref, xh_ref, rn_ref):
    xx = x_ref[...]
    xa_ref[...] = xx[:, :H]
    xb_ref[...] = xx[:, H:]
    xh_ref[...] = xx.astype(jnp.bfloat16)
    ss = jnp.sum(xx * xx, axis=1, keepdims=True)
    rn_ref[...] = 1.0 / jnp.maximum(jnp.sqrt(ss), 1e-12)


_prep = pl.pallas_call(
    _prep_body,
    out_shape=[
        jax.ShapeDtypeStruct((N, H), jnp.float32),
        jax.ShapeDtypeStruct((N, H), jnp.float32),
        jax.ShapeDtypeStruct((N, D), jnp.bfloat16),
        jax.ShapeDtypeStruct((N, 1), jnp.float32),
    ],
)


def _as_i32_pairs(xh):
    return jax.lax.bitcast_convert_type(xh.reshape(N, H, 2), jnp.int32)


def _score_body(x_hbm, row_hbm, col_hbm, rn_hbm, beta_hbm, zeros_hbm,
                w_hbm, se_hbm, *sc):
    b0, b1 = sc[:13], sc[13:26]
    rn_v, beta_v, se_tmp_v, se_sh = sc[26:]
    c = jax.lax.axis_index("c")
    s = jax.lax.axis_index("s")
    wid = c * NS + s
    pltpu.sync_copy(rn_hbm, rn_v)
    pltpu.sync_copy(beta_hbm, beta_v)

    @pl.when(s == 0)
    def _():
        pltpu.sync_copy(zeros_hbm, se_sh)

    plsc.subcore_barrier()

    beta_vec = beta_v[...]
    lanes = jax.lax.iota(jnp.int32, L)
    last_lane = lanes == (L - 1)
    nch = (NCHUNKS - wid + 31) // 32

    def chunk_base(q):
        return (wid + q * 32) * CHUNK

    def idx_start(q, b):
        idxr, idxc = b[0], b[1]
        s_ir, s_ic = b[5], b[6]
        base = chunk_base(q)
        pltpu.async_copy(row_hbm.at[pl.ds(base, CHUNK)], idxr, s_ir)
        pltpu.async_copy(col_hbm.at[pl.ds(base, CHUNK)], idxc, s_ic)

    def idx_wait(b):
        idxr, idxc = b[0], b[1]
        s_ir, s_ic = b[5], b[6]
        pltpu.make_async_copy(row_hbm.at[pl.ds(0, CHUNK)], idxr, s_ir).wait()
        pltpu.make_async_copy(col_hbm.at[pl.ds(0, CHUNK)], idxc, s_ic).wait()

    def gather_start(b):
        idxr, idxc, xr, xc = b[0], b[1], b[2], b[3]
        s_gr, s_gc = b[7], b[8]
        pltpu.async_copy(x_hbm.at[idxr], xr, s_gr)
        pltpu.async_copy(x_hbm.at[idxc], xc, s_gc)

    def gather_wait(b):
        idxr, idxc, xr, xc = b[0], b[1], b[2], b[3]
        s_gr, s_gc = b[7], b[8]
        pltpu.make_async_copy(x_hbm.at[idxr], xr, s_gr).wait()
        pltpu.make_async_copy(x_hbm.at[idxc], xc, s_gc).wait()

    def out_start(j, b):
        w_v, idxs_c = b[4], b[12]
        s_w, s_se = b[9], b[10]
        pltpu.async_copy(w_v, w_hbm.at[pl.ds(chunk_base(j), CHUNK)], s_w)
        pltpu.async_copy(w_v, se_sh.at[idxs_c], s_se, add=True)

    def out_wait(b):
        w_v, idxs_c = b[4], b[12]
        s_w, s_se = b[9], b[10]
        pltpu.make_async_copy(w_v, w_hbm.at[pl.ds(0, CHUNK)], s_w).wait()
        pltpu.make_async_copy(w_v, se_sh.at[idxs_c], s_se).wait()

    def snapshot(b):
        idxr, idxc = b[0], b[1]
        idxs_r, idxs_c = b[11], b[12]
        for g in range(CHUNK // L):
            sl = pl.ds(g * L, L)
            idxs_r[sl] = idxr[sl]
            idxs_c[sl] = idxc[sl]

    def compute(b):
        xr_v, xc_v, w_v = b[2], b[3], b[4]
        idxs_r, idxs_c = b[11], b[12]

        @pl.loop(0, CHUNK, unroll=4)
        def _edge(e):
            acc = None
            for k in range(D // (2 * L)):
                a = plsc.bitcast(xr_v[e, pl.ds(k * L, L)], jnp.bfloat16)
                bb = plsc.bitcast(xc_v[e, pl.ds(k * L, L)], jnp.bfloat16)
                u0, u1 = plsc.unpack(a * bb,
                                     format=plsc.PackFormat.INTERLEAVED)
                acc = u0 + u1 if acc is None else acc + u0 + u1
            sc_ = plsc.cumsum(acc)
            plsc.store_scatter(w_v, [jnp.broadcast_to(e, (L,))], sc_,
                               mask=last_lane)

        for g in range(CHUNK // L):
            rr = plsc.load_gather(rn_v, [idxs_r[pl.ds(g * L, L)]])
            rc = plsc.load_gather(rn_v, [idxs_c[pl.ds(g * L, L)]])
            dots = w_v[pl.ds(g * L, L)]
            w_v[pl.ds(g * L, L)] = jnp.exp(dots * rr * rc * beta_vec)

    def step(j, b, bn):
        gather_wait(b)

        @pl.when(j + 1 < nch)
        def _():
            idx_wait(bn)
            gather_start(bn)

        @pl.when(j >= 2)
        def _():
            out_wait(b)

        snapshot(b)

        @pl.when(j + 2 < nch)
        def _():
            idx_start(j + 2, b)

        compute(b)
        out_start(j, b)

    idx_start(0, b0)
    idx_start(1, b1)
    idx_wait(b0)
    gather_start(b0)

    @pl.loop(0, nch)
    def _chunk(j):
        @pl.when(j % 2 == 0)
        def _():
            step(j, b0, b1)

        @pl.when(j % 2 == 1)
        def _():
            step(j, b1, b0)

    out_wait(b0)
    out_wait(b1)
    plsc.subcore_barrier()

    @pl.when(s == 0)
    def _():
        pltpu.sync_copy(se_sh, se_tmp_v)
        pltpu.sync_copy(se_tmp_v, se_hbm.at[pl.ds(c * N, N)])


_score = functools.partial(
    pl.kernel,
    out_type=[
        jax.ShapeDtypeStruct((E,), jnp.float32),
        jax.ShapeDtypeStruct((NC * N,), jnp.float32),
    ],
    mesh=_mesh,
    compiler_params=_sc_params,
    scratch_types=(
        2 * [
            pltpu.VMEM((CHUNK,), jnp.int32),
            pltpu.VMEM((CHUNK,), jnp.int32),
            pltpu.VMEM((CHUNK, H), jnp.int32),
            pltpu.VMEM((CHUNK, H), jnp.int32),
            pltpu.VMEM((CHUNK,), jnp.float32),
            pltpu.SemaphoreType.DMA,
            pltpu.SemaphoreType.DMA,
            pltpu.SemaphoreType.DMA,
            pltpu.SemaphoreType.DMA,
            pltpu.SemaphoreType.DMA,
            pltpu.SemaphoreType.DMA,
            pltpu.VMEM((CHUNK,), jnp.int32),
            pltpu.VMEM((CHUNK,), jnp.int32),
        ]
        + [
            pltpu.VMEM((N,), jnp.float32),
            pltpu.VMEM((L,), jnp.float32),
            pltpu.VMEM((N,), jnp.float32),
            pltpu.VMEM_SHARED((N,), jnp.float32),
        ]
    ),
)(_score_body)


CCHUNK = 160
NCCH = E // CCHUNK


def _scatter_body(xa_hbm, xb_hbm, row_hbm, col_hbm, w_hbm, zacc_hbm,
                  acc_hbm, *sc):
    b0, b1 = sc[:9], sc[9:18]
    (acc_sh,) = sc[18:]
    c = jax.lax.axis_index("c")
    s = jax.lax.axis_index("s")

    @pl.loop(0, (NRCH - s + NS - 1) // NS)
    def _zero(t):
        r0 = (s + t * NS) * RCHUNK
        pltpu.sync_copy(zacc_hbm.at[pl.ds(r0, RCHUNK)],
                        acc_sh.at[pl.ds(r0, RCHUNK)])

    @pl.when(s == 0)
    def _():
        pltpu.sync_copy(zacc_hbm.at[pl.ds(NRCH * RCHUNK, RTAIL)],
                        acc_sh.at[pl.ds(NRCH * RCHUNK, RTAIL)])

    plsc.subcore_barrier()

    nch = (NCCH - s + 15) // 16

    def chunk_base(q):
        return (s + q * 16) * CCHUNK

    def idx_start(q, b):
        idxr, idxc, w_v = b[0], b[1], b[2]
        s_ir, s_ic, s_iw = b[4], b[5], b[6]
        base = chunk_base(q)
        pltpu.async_copy(row_hbm.at[pl.ds(base, CCHUNK)], idxr, s_ir)
        pltpu.async_copy(col_hbm.at[pl.ds(base, CCHUNK)], idxc, s_ic)
        pltpu.async_copy(w_hbm.at[pl.ds(base, CCHUNK)], w_v, s_iw)

    def idx_wait(b):
        idxr, idxc, w_v = b[0], b[1], b[2]
        s_ir, s_ic, s_iw = b[4], b[5], b[6]
        pltpu.make_async_copy(row_hbm.at[pl.ds(0, CCHUNK)], idxr, s_ir).wait()
        pltpu.make_async_copy(col_hbm.at[pl.ds(0, CCHUNK)], idxc, s_ic).wait()
        pltpu.make_async_copy(w_hbm.at[pl.ds(0, CCHUNK)], w_v, s_iw).wait()

    def gather_start(b):
        idxr, rows_v, s_g = b[0], b[3], b[7]

        @pl.when(c == 0)
        def _():
            pltpu.async_copy(xa_hbm.at[idxr], rows_v, s_g)

        @pl.when(c == 1)
        def _():
            pltpu.async_copy(xb_hbm.at[idxr], rows_v, s_g)

    def gather_wait(b):
        idxr, rows_v, s_g = b[0], b[3], b[7]

        @pl.when(c == 0)
        def _():
            pltpu.make_async_copy(xa_hbm.at[idxr], rows_v, s_g).wait()

        @pl.when(c == 1)
        def _():
            pltpu.make_async_copy(xb_hbm.at[idxr], rows_v, s_g).wait()

    def scatter_start(b):
        idxc, rows_v, s_sc = b[1], b[3], b[8]
        pltpu.async_copy(rows_v, acc_sh.at[idxc], s_sc, add=True)

    def scatter_wait(b):
        idxc, rows_v, s_sc = b[1], b[3], b[8]
        pltpu.make_async_copy(rows_v, acc_sh.at[idxc], s_sc).wait()

    def multiply(b):
        w_v, rows_v = b[2], b[3]

        @pl.loop(0, CCHUNK, unroll=4)
        def _edge(e):
            ws = plsc.load_gather(w_v, [jnp.broadcast_to(e, (L,))])
            for k in range(H // L):
                sl = (e, pl.ds(k * L, L))
                rows_v[sl] = rows_v[sl] * ws

    def step(j, b, bn):
        gather_wait(b)

        @pl.when(j + 1 < nch)
        def _():
            @pl.when(j >= 1)
            def _():
                scatter_wait(bn)
            idx_start(j + 1, bn)

        multiply(b)
        scatter_start(b)

        @pl.when(j + 1 < nch)
        def _():
            idx_wait(bn)
            gather_start(bn)

    idx_start(0, b0)
    idx_wait(b0)
    gather_start(b0)

    @pl.loop(0, nch)
    def _chunk(j):
        @pl.when(j % 2 == 0)
        def _():
            step(j, b0, b1)

        @pl.when(j % 2 == 1)
        def _():
            step(j, b1, b0)

    scatter_wait(b0)
    scatter_wait(b1)

    plsc.subcore_barrier()

    rows0, rows1 = b0[3], b1[3]

    @pl.loop(0, (NRCH - s + NS - 1) // NS)
    def _drain(t):
        r0 = (s + t * NS) * RCHUNK
        pltpu.sync_copy(acc_sh.at[pl.ds(r0, RCHUNK)], rows0.at[pl.ds(0, RCHUNK)])
        pltpu.sync_copy(rows0.at[pl.ds(0, RCHUNK)],
                        acc_hbm.at[c].at[pl.ds(r0, RCHUNK)])

    @pl.when(s == 0)
    def _():
        r0 = NRCH * RCHUNK
        pltpu.sync_copy(acc_sh.at[pl.ds(r0, RTAIL)], rows1.at[pl.ds(0, RTAIL)])
        pltpu.sync_copy(rows1.at[pl.ds(0, RTAIL)],
                        acc_hbm.at[c].at[pl.ds(r0, RTAIL)])


_scatter = functools.partial(
    pl.kernel,
    out_type=jax.ShapeDtypeStruct((NC, N, H), jnp.float32),
    mesh=_mesh,
    compiler_params=_sc_params,
    scratch_types=(
        2 * [
            pltpu.VMEM((CCHUNK,), jnp.int32),
            pltpu.VMEM((CCHUNK,), jnp.int32),
            pltpu.VMEM((CCHUNK,), jnp.float32),
            pltpu.VMEM((CCHUNK, H), jnp.float32),
            pltpu.SemaphoreType.DMA,
            pltpu.SemaphoreType.DMA,
            pltpu.SemaphoreType.DMA,
            pltpu.SemaphoreType.DMA,
            pltpu.SemaphoreType.DMA,
        ]
        + [pltpu.VMEM_SHARED((N, H), jnp.float32)]
    ),
)(_scatter_body)


def _final_body(acca_ref, accb_ref, s0_ref, s1_ref, out_ref):
    inv = 1.0 / jnp.maximum(s0_ref[...] + s1_ref[...], 1e-16)
    out_ref[:, :H] = acca_ref[...] * inv
    out_ref[:, H:] = accb_ref[...] * inv


_final = pl.pallas_call(
    _final_body,
    out_shape=jax.ShapeDtypeStruct((N, D), jnp.float32),
)


def kernel(x, edge_index, beta):
    x = x.astype(jnp.float32)
    row = edge_index[0].astype(jnp.int32)
    col = edge_index[1].astype(jnp.int32)
    beta16 = jnp.broadcast_to(beta.astype(jnp.float32), (L,))
    zeros_n = jnp.zeros((N,), jnp.float32)
    zacc = jnp.zeros((N, H), jnp.float32)

    xa, xb, xh, rn2 = _prep(x)
    rn = rn2.reshape(N)
    w, sumexp = _score(_as_i32_pairs(xh), row, col, rn, beta16, zeros_n)
    acc = _scatter(xa, xb, row, col, w, zacc)
    out = _final(acc[0], acc[1],
                 sumexp[:N].reshape(N, 1), sumexp[N:].reshape(N, 1))
    return out


# speedup vs baseline: 1.1849x; 1.1849x over previous
import dataclasses
import functools

import jax
import jax.numpy as jnp
from jax.experimental import pallas as pl
from jax.experimental.pallas import tpu as pltpu
from jax.experimental.pallas import tpu_sc as plsc

N = 10000
E = 160000
D = 256
H = D // 2
NC = 2
NS = 16
L = 16
CHUNK = 64
NCHUNKS = E // CHUNK
RCHUNK = 128
NRCH = N // RCHUNK
RTAIL = N - NRCH * RCHUNK

_mesh = plsc.VectorSubcoreMesh(
    core_axis_name="c", subcore_axis_name="s", num_cores=NC, num_subcores=NS
)

_sc_params = pltpu.CompilerParams()
if "needs_layout_passes" in pltpu.CompilerParams.__dataclass_fields__:
    _sc_params = dataclasses.replace(_sc_params, needs_layout_passes=False)


def _prep_body(x_ref, xa_ref, xb_ref, rn_ref):
    xx = x_ref[...]
    xa_ref[...] = xx[:, :H]
    xb_ref[...] = xx[:, H:]
    ss = jnp.sum(xx * xx, axis=1, keepdims=True)
    rn_ref[...] = 1.0 / jnp.maximum(jnp.sqrt(ss), 1e-12)


_prep = pl.pallas_call(
    _prep_body,
    out_shape=[
        jax.ShapeDtypeStruct((N, H), jnp.float32),
        jax.ShapeDtypeStruct((N, H), jnp.float32),
        jax.ShapeDtypeStruct((N, 1), jnp.float32),
    ],
)


def _score_body(x_hbm, row_hbm, col_hbm, rn_hbm, beta_hbm, zeros_hbm,
                w_hbm, se_hbm, *sc):
    b0, b1 = sc[:13], sc[13:26]
    rn_v, beta_v, se_tmp_v, se_sh = sc[26:]
    c = jax.lax.axis_index("c")
    s = jax.lax.axis_index("s")
    wid = c * NS + s
    pltpu.sync_copy(rn_hbm, rn_v)
    pltpu.sync_copy(beta_hbm, beta_v)

    @pl.when(s == 0)
    def _():
        pltpu.sync_copy(zeros_hbm, se_sh)

    plsc.subcore_barrier()

    beta_vec = beta_v[...]
    lanes = jax.lax.iota(jnp.int32, L)
    last_lane = lanes == (L - 1)
    nch = (NCHUNKS - wid + 31) // 32

    def chunk_base(q):
        return (wid + q * 32) * CHUNK

    def idx_start(q, b):
        idxr, idxc = b[0], b[1]
        s_ir, s_ic = b[5], b[6]
        base = chunk_base(q)
        pltpu.async_copy(row_hbm.at[pl.ds(base, CHUNK)], idxr, s_ir)
        pltpu.async_copy(col_hbm.at[pl.ds(base, CHUNK)], idxc, s_ic)

    def idx_wait(b):
        idxr, idxc = b[0], b[1]
        s_ir, s_ic = b[5], b[6]
        pltpu.make_async_copy(row_hbm.at[pl.ds(0, CHUNK)], idxr, s_ir).wait()
        pltpu.make_async_copy(col_hbm.at[pl.ds(0, CHUNK)], idxc, s_ic).wait()

    def gather_start(b):
        idxr, idxc, xr, xc = b[0], b[1], b[2], b[3]
        s_gr, s_gc = b[7], b[8]
        pltpu.async_copy(x_hbm.at[idxr], xr, s_gr)
        pltpu.async_copy(x_hbm.at[idxc], xc, s_gc)

    def gather_wait(b):
        idxr, idxc, xr, xc = b[0], b[1], b[2], b[3]
        s_gr, s_gc = b[7], b[8]
        pltpu.make_async_copy(x_hbm.at[idxr], xr, s_gr).wait()
        pltpu.make_async_copy(x_hbm.at[idxc], xc, s_gc).wait()

    def out_start(j, b):
        w_v, idxs_c = b[4], b[12]
        s_w, s_se = b[9], b[10]
        pltpu.async_copy(w_v, w_hbm.at[pl.ds(chunk_base(j), CHUNK)], s_w)
        pltpu.async_copy(w_v, se_sh.at[idxs_c], s_se, add=True)

    def out_wait(b):
        w_v, idxs_c = b[4], b[12]
        s_w, s_se = b[9], b[10]
        pltpu.make_async_copy(w_v, w_hbm.at[pl.ds(0, CHUNK)], s_w).wait()
        pltpu.make_async_copy(w_v, se_sh.at[idxs_c], s_se).wait()

    def snapshot(b):
        idxr, idxc = b[0], b[1]
        idxs_r, idxs_c = b[11], b[12]
        for g in range(CHUNK // L):
            sl = pl.ds(g * L, L)
            idxs_r[sl] = idxr[sl]
            idxs_c[sl] = idxc[sl]

    def compute(b):
        xr_v, xc_v, w_v = b[2], b[3], b[4]
        idxs_r, idxs_c = b[11], b[12]

        @pl.loop(0, CHUNK, unroll=4)
        def _edge(e):
            acc = xr_v[e, pl.ds(0, L)] * xc_v[e, pl.ds(0, L)]
            for k in range(1, D // L):
                acc += xr_v[e, pl.ds(k * L, L)] * xc_v[e, pl.ds(k * L, L)]
            sc_ = plsc.cumsum(acc)
            plsc.store_scatter(w_v, [jnp.broadcast_to(e, (L,))], sc_,
                               mask=last_lane)

        for g in range(CHUNK // L):
            rr = plsc.load_gather(rn_v, [idxs_r[pl.ds(g * L, L)]])
            rc = plsc.load_gather(rn_v, [idxs_c[pl.ds(g * L, L)]])
            dots = w_v[pl.ds(g * L, L)]
            w_v[pl.ds(g * L, L)] = jnp.exp(dots * rr * rc * beta_vec)

    def step(j, b, bn):
        gather_wait(b)

        @pl.when(j + 1 < nch)
        def _():
            idx_wait(bn)
            gather_start(bn)

        @pl.when(j >= 2)
        def _():
            out_wait(b)

        snapshot(b)

        @pl.when(j + 2 < nch)
        def _():
            idx_start(j + 2, b)

        compute(b)
        out_start(j, b)

    idx_start(0, b0)
    idx_start(1, b1)
    idx_wait(b0)
    gather_start(b0)

    @pl.loop(0, nch)
    def _chunk(j):
        @pl.when(j % 2 == 0)
        def _():
            step(j, b0, b1)

        @pl.when(j % 2 == 1)
        def _():
            step(j, b1, b0)

    out_wait(b0)
    out_wait(b1)
    plsc.subcore_barrier()

    @pl.when(s == 0)
    def _():
        pltpu.sync_copy(se_sh, se_tmp_v)
        pltpu.sync_copy(se_tmp_v, se_hbm.at[pl.ds(c * N, N)])


_score = functools.partial(
    pl.kernel,
    out_type=[
        jax.ShapeDtypeStruct((E,), jnp.float32),
        jax.ShapeDtypeStruct((NC * N,), jnp.float32),
    ],
    mesh=_mesh,
    compiler_params=_sc_params,
    scratch_types=(
        2 * [
            pltpu.VMEM((CHUNK,), jnp.int32),
            pltpu.VMEM((CHUNK,), jnp.int32),
            pltpu.VMEM((CHUNK, D), jnp.float32),
            pltpu.VMEM((CHUNK, D), jnp.float32),
            pltpu.VMEM((CHUNK,), jnp.float32),
            pltpu.SemaphoreType.DMA,
            pltpu.SemaphoreType.DMA,
            pltpu.SemaphoreType.DMA,
            pltpu.SemaphoreType.DMA,
            pltpu.SemaphoreType.DMA,
            pltpu.SemaphoreType.DMA,
            pltpu.VMEM((CHUNK,), jnp.int32),
            pltpu.VMEM((CHUNK,), jnp.int32),
        ]
        + [
            pltpu.VMEM((N,), jnp.float32),
            pltpu.VMEM((L,), jnp.float32),
            pltpu.VMEM((N,), jnp.float32),
            pltpu.VMEM_SHARED((N,), jnp.float32),
        ]
    ),
)(_score_body)


CCHUNK = 160
NCCH = E // CCHUNK


def _scatter_body(xa_hbm, xb_hbm, row_hbm, col_hbm, w_hbm, zacc_hbm,
                  acc_hbm, *sc):
    b0, b1 = sc[:9], sc[9:18]
    (acc_sh,) = sc[18:]
    c = jax.lax.axis_index("c")
    s = jax.lax.axis_index("s")

    @pl.loop(0, (NRCH - s + NS - 1) // NS)
    def _zero(t):
        r0 = (s + t * NS) * RCHUNK
        pltpu.sync_copy(zacc_hbm.at[pl.ds(r0, RCHUNK)],
                        acc_sh.at[pl.ds(r0, RCHUNK)])

    @pl.when(s == 0)
    def _():
        pltpu.sync_copy(zacc_hbm.at[pl.ds(NRCH * RCHUNK, RTAIL)],
                        acc_sh.at[pl.ds(NRCH * RCHUNK, RTAIL)])

    plsc.subcore_barrier()

    nch = (NCCH - s + 15) // 16

    def chunk_base(q):
        return (s + q * 16) * CCHUNK

    def idx_start(q, b):
        idxr, idxc, w_v = b[0], b[1], b[2]
        s_ir, s_ic, s_iw = b[4], b[5], b[6]
        base = chunk_base(q)
        pltpu.async_copy(row_hbm.at[pl.ds(base, CCHUNK)], idxr, s_ir)
        pltpu.async_copy(col_hbm.at[pl.ds(base, CCHUNK)], idxc, s_ic)
        pltpu.async_copy(w_hbm.at[pl.ds(base, CCHUNK)], w_v, s_iw)

    def idx_wait(b):
        idxr, idxc, w_v = b[0], b[1], b[2]
        s_ir, s_ic, s_iw = b[4], b[5], b[6]
        pltpu.make_async_copy(row_hbm.at[pl.ds(0, CCHUNK)], idxr, s_ir).wait()
        pltpu.make_async_copy(col_hbm.at[pl.ds(0, CCHUNK)], idxc, s_ic).wait()
        pltpu.make_async_copy(w_hbm.at[pl.ds(0, CCHUNK)], w_v, s_iw).wait()

    def gather_start(b):
        idxr, rows_v, s_g = b[0], b[3], b[7]

        @pl.when(c == 0)
        def _():
            pltpu.async_copy(xa_hbm.at[idxr], rows_v, s_g)

        @pl.when(c == 1)
        def _():
            pltpu.async_copy(xb_hbm.at[idxr], rows_v, s_g)

    def gather_wait(b):
        idxr, rows_v, s_g = b[0], b[3], b[7]

        @pl.when(c == 0)
        def _():
            pltpu.make_async_copy(xa_hbm.at[idxr], rows_v, s_g).wait()

        @pl.when(c == 1)
        def _():
            pltpu.make_async_copy(xb_hbm.at[idxr], rows_v, s_g).wait()

    def scatter_start(b):
        idxc, rows_v, s_sc = b[1], b[3], b[8]
        pltpu.async_copy(rows_v, acc_sh.at[idxc], s_sc, add=True)

    def scatter_wait(b):
        idxc, rows_v, s_sc = b[1], b[3], b[8]
        pltpu.make_async_copy(rows_v, acc_sh.at[idxc], s_sc).wait()

    def multiply(b):
        w_v, rows_v = b[2], b[3]

        @pl.loop(0, CCHUNK, unroll=4)
        def _edge(e):
            ws = plsc.load_gather(w_v, [jnp.broadcast_to(e, (L,))])
            for k in range(H // L):
                sl = (e, pl.ds(k * L, L))
                rows_v[sl] = rows_v[sl] * ws

    def step(j, b, bn):
        gather_wait(b)

        @pl.when(j + 1 < nch)
        def _():
            @pl.when(j >= 1)
            def _():
                scatter_wait(bn)
            idx_start(j + 1, bn)

        multiply(b)
        scatter_start(b)

        @pl.when(j + 1 < nch)
        def _():
            idx_wait(bn)
            gather_start(bn)

    idx_start(0, b0)
    idx_wait(b0)
    gather_start(b0)

    @pl.loop(0, nch)
    def _chunk(j):
        @pl.when(j % 2 == 0)
        def _():
            step(j, b0, b1)

        @pl.when(j % 2 == 1)
        def _():
            step(j, b1, b0)

    scatter_wait(b0)
    scatter_wait(b1)

    plsc.subcore_barrier()

    rows0, rows1 = b0[3], b1[3]

    @pl.loop(0, (NRCH - s + NS - 1) // NS)
    def _drain(t):
        r0 = (s + t * NS) * RCHUNK
        pltpu.sync_copy(acc_sh.at[pl.ds(r0, RCHUNK)], rows0.at[pl.ds(0, RCHUNK)])
        pltpu.sync_copy(rows0.at[pl.ds(0, RCHUNK)],
                        acc_hbm.at[c].at[pl.ds(r0, RCHUNK)])

    @pl.when(s == 0)
    def _():
        r0 = NRCH * RCHUNK
        pltpu.sync_copy(acc_sh.at[pl.ds(r0, RTAIL)], rows1.at[pl.ds(0, RTAIL)])
        pltpu.sync_copy(rows1.at[pl.ds(0, RTAIL)],
                        acc_hbm.at[c].at[pl.ds(r0, RTAIL)])


_scatter = functools.partial(
    pl.kernel,
    out_type=jax.ShapeDtypeStruct((NC, N, H), jnp.float32),
    mesh=_mesh,
    compiler_params=_sc_params,
    scratch_types=(
        2 * [
            pltpu.VMEM((CCHUNK,), jnp.int32),
            pltpu.VMEM((CCHUNK,), jnp.int32),
            pltpu.VMEM((CCHUNK,), jnp.float32),
            pltpu.VMEM((CCHUNK, H), jnp.float32),
            pltpu.SemaphoreType.DMA,
            pltpu.SemaphoreType.DMA,
            pltpu.SemaphoreType.DMA,
            pltpu.SemaphoreType.DMA,
            pltpu.SemaphoreType.DMA,
        ]
        + [pltpu.VMEM_SHARED((N, H), jnp.float32)]
    ),
)(_scatter_body)


def _final_body(acca_ref, accb_ref, s0_ref, s1_ref, out_ref):
    inv = 1.0 / jnp.maximum(s0_ref[...] + s1_ref[...], 1e-16)
    out_ref[:, :H] = acca_ref[...] * inv
    out_ref[:, H:] = accb_ref[...] * inv


_final = pl.pallas_call(
    _final_body,
    out_shape=jax.ShapeDtypeStruct((N, D), jnp.float32),
)


def kernel(x, edge_index, beta):
    x = x.astype(jnp.float32)
    row = edge_index[0].astype(jnp.int32)
    col = edge_index[1].astype(jnp.int32)
    beta16 = jnp.broadcast_to(beta.astype(jnp.float32), (L,))
    zeros_n = jnp.zeros((N,), jnp.float32)
    zacc = jnp.zeros((N, H), jnp.float32)

    xa, xb, rn2 = _prep(x)
    rn = rn2.reshape(N)
    w, sumexp = _score(x, row, col, rn, beta16, zeros_n)
    acc = _scatter(xa, xb, row, col, w, zacc)
    out = _final(acc[0], acc[1],
                 sumexp[:N].reshape(N, 1), sumexp[N:].reshape(N, 1))
    return out


# speedup vs baseline: 1.1998x; 1.0125x over previous
import dataclasses
import functools

import jax
import jax.numpy as jnp
from jax.experimental import pallas as pl
from jax.experimental.pallas import tpu as pltpu
from jax.experimental.pallas import tpu_sc as plsc

N = 10000
E = 160000
D = 256
H = D // 2
NC = 2
NS = 16
L = 16
CHUNK = 64
NCHUNKS = E // CHUNK
RCHUNK = 128
NRCH = N // RCHUNK
RTAIL = N - NRCH * RCHUNK

_mesh = plsc.VectorSubcoreMesh(
    core_axis_name="c", subcore_axis_name="s", num_cores=NC, num_subcores=NS
)

_sc_params = pltpu.CompilerParams()
if "needs_layout_passes" in pltpu.CompilerParams.__dataclass_fields__:
    _sc_params = dataclasses.replace(_sc_params, needs_layout_passes=False)


def _prep_body(x_ref, xa_ref, xb_ref, rn_ref):
    xx = x_ref[...]
    xa_ref[...] = xx[:, :H]
    xb_ref[...] = xx[:, H:]
    ss = jnp.sum(xx * xx, axis=1, keepdims=True)
    rn_ref[...] = 1.0 / jnp.maximum(jnp.sqrt(ss), 1e-12)


_prep = pl.pallas_call(
    _prep_body,
    out_shape=[
        jax.ShapeDtypeStruct((N, H), jnp.float32),
        jax.ShapeDtypeStruct((N, H), jnp.float32),
        jax.ShapeDtypeStruct((N, 1), jnp.float32),
    ],
)


def _score_body(x_hbm, row_hbm, col_hbm, rn_hbm, beta_hbm, zeros_hbm,
                w_hbm, se_hbm, *sc):
    b0, b1 = sc[:13], sc[13:26]
    rn_v, beta_v, se_tmp_v, se_sh = sc[26:]
    c = jax.lax.axis_index("c")
    s = jax.lax.axis_index("s")
    wid = c * NS + s
    pltpu.sync_copy(rn_hbm, rn_v)
    pltpu.sync_copy(beta_hbm, beta_v)

    @pl.when(s == 0)
    def _():
        pltpu.sync_copy(zeros_hbm, se_sh)

    plsc.subcore_barrier()

    beta_vec = beta_v[...]
    lanes = jax.lax.iota(jnp.int32, L)
    last_lane = lanes == (L - 1)
    nch = (NCHUNKS - wid + 31) // 32

    def chunk_base(q):
        return (wid + q * 32) * CHUNK

    def idx_start(q, b):
        idxr, idxc = b[0], b[1]
        s_ir, s_ic = b[5], b[6]
        base = chunk_base(q)
        pltpu.async_copy(row_hbm.at[pl.ds(base, CHUNK)], idxr, s_ir)
        pltpu.async_copy(col_hbm.at[pl.ds(base, CHUNK)], idxc, s_ic)

    def idx_wait(b):
        idxr, idxc = b[0], b[1]
        s_ir, s_ic = b[5], b[6]
        pltpu.make_async_copy(row_hbm.at[pl.ds(0, CHUNK)], idxr, s_ir).wait()
        pltpu.make_async_copy(col_hbm.at[pl.ds(0, CHUNK)], idxc, s_ic).wait()

    def gather_start(b):
        idxr, idxc, xr, xc = b[0], b[1], b[2], b[3]
        s_gr, s_gc = b[7], b[8]
        pltpu.async_copy(x_hbm.at[idxr], xr, s_gr)
        pltpu.async_copy(x_hbm.at[idxc], xc, s_gc)

    def gather_wait(b):
        idxr, idxc, xr, xc = b[0], b[1], b[2], b[3]
        s_gr, s_gc = b[7], b[8]
        pltpu.make_async_copy(x_hbm.at[idxr], xr, s_gr).wait()
        pltpu.make_async_copy(x_hbm.at[idxc], xc, s_gc).wait()

    def out_start(j, b):
        w_v, idxs_c = b[4], b[12]
        s_w, s_se = b[9], b[10]
        pltpu.async_copy(w_v, w_hbm.at[pl.ds(chunk_base(j), CHUNK)], s_w)
        pltpu.async_copy(w_v, se_sh.at[idxs_c], s_se, add=True)

    def out_wait(b):
        w_v, idxs_c = b[4], b[12]
        s_w, s_se = b[9], b[10]
        pltpu.make_async_copy(w_v, w_hbm.at[pl.ds(0, CHUNK)], s_w).wait()
        pltpu.make_async_copy(w_v, se_sh.at[idxs_c], s_se).wait()

    def snapshot(b):
        idxr, idxc = b[0], b[1]
        idxs_r, idxs_c = b[11], b[12]
        for g in range(CHUNK // L):
            sl = pl.ds(g * L, L)
            idxs_r[sl] = idxr[sl]
            idxs_c[sl] = idxc[sl]

    def compute(b):
        xr_v, xc_v, w_v = b[2], b[3], b[4]
        idxs_r, idxs_c = b[11], b[12]

        @pl.loop(0, CHUNK, unroll=4)
        def _edge(e):
            acc = xr_v[e, pl.ds(0, L)] * xc_v[e, pl.ds(0, L)]
            for k in range(1, D // L):
                acc += xr_v[e, pl.ds(k * L, L)] * xc_v[e, pl.ds(k * L, L)]
            sc_ = plsc.cumsum(acc)
            plsc.store_scatter(w_v, [jnp.broadcast_to(e, (L,))], sc_,
                               mask=last_lane)

        for g in range(CHUNK // L):
            rr = plsc.load_gather(rn_v, [idxs_r[pl.ds(g * L, L)]])
            rc = plsc.load_gather(rn_v, [idxs_c[pl.ds(g * L, L)]])
            dots = w_v[pl.ds(g * L, L)]
            w_v[pl.ds(g * L, L)] = jnp.exp(dots * rr * rc * beta_vec)

    def step(j, b, bn):
        gather_wait(b)

        @pl.when(j + 1 < nch)
        def _():
            idx_wait(bn)
            gather_start(bn)

        @pl.when(j >= 2)
        def _():
            out_wait(b)

        snapshot(b)

        @pl.when(j + 2 < nch)
        def _():
            idx_start(j + 2, b)

        compute(b)
        out_start(j, b)

    idx_start(0, b0)
    idx_start(1, b1)
    idx_wait(b0)
    gather_start(b0)

    @pl.loop(0, nch)
    def _chunk(j):
        @pl.when(j % 2 == 0)
        def _():
            step(j, b0, b1)

        @pl.when(j % 2 == 1)
        def _():
            step(j, b1, b0)

    out_wait(b0)
    out_wait(b1)
    plsc.subcore_barrier()

    @pl.when(s == 0)
    def _():
        pltpu.sync_copy(se_sh, se_tmp_v)
        pltpu.sync_copy(se_tmp_v, se_hbm.at[pl.ds(c * N, N)])


_score = functools.partial(
    pl.kernel,
    out_type=[
        jax.ShapeDtypeStruct((E,), jnp.float32),
        jax.ShapeDtypeStruct((NC * N,), jnp.float32),
    ],
    mesh=_mesh,
    compiler_params=_sc_params,
    scratch_types=(
        2 * [
            pltpu.VMEM((CHUNK,), jnp.int32),
            pltpu.VMEM((CHUNK,), jnp.int32),
            pltpu.VMEM((CHUNK, D), jnp.float32),
            pltpu.VMEM((CHUNK, D), jnp.float32),
            pltpu.VMEM((CHUNK,), jnp.float32),
            pltpu.SemaphoreType.DMA,
            pltpu.SemaphoreType.DMA,
            pltpu.SemaphoreType.DMA,
            pltpu.SemaphoreType.DMA,
            pltpu.SemaphoreType.DMA,
            pltpu.SemaphoreType.DMA,
            pltpu.VMEM((CHUNK,), jnp.int32),
            pltpu.VMEM((CHUNK,), jnp.int32),
        ]
        + [
            pltpu.VMEM((N,), jnp.float32),
            pltpu.VMEM((L,), jnp.float32),
            pltpu.VMEM((N,), jnp.float32),
            pltpu.VMEM_SHARED((N,), jnp.float32),
        ]
    ),
)(_score_body)


CCHUNK = 160
NCCH = E // CCHUNK


def _scatter_body(xa_hbm, xb_hbm, row_hbm, col_hbm, w_hbm, zacc_hbm, se_hbm,
                  out_hbm, *sc):
    b0, b1 = sc[:9], sc[9:18]
    acc_sh, se0_v, se1_v, inv_v = sc[18:]
    c = jax.lax.axis_index("c")
    s = jax.lax.axis_index("s")

    @pl.loop(0, (NRCH - s + NS - 1) // NS)
    def _zero(t):
        r0 = (s + t * NS) * RCHUNK
        pltpu.sync_copy(zacc_hbm.at[pl.ds(r0, RCHUNK)],
                        acc_sh.at[pl.ds(r0, RCHUNK)])

    @pl.when(s == 0)
    def _():
        pltpu.sync_copy(zacc_hbm.at[pl.ds(NRCH * RCHUNK, RTAIL)],
                        acc_sh.at[pl.ds(NRCH * RCHUNK, RTAIL)])

    plsc.subcore_barrier()

    nch = (NCCH - s + 15) // 16

    def chunk_base(q):
        return (s + q * 16) * CCHUNK

    def idx_start(q, b):
        idxr, idxc, w_v = b[0], b[1], b[2]
        s_ir, s_ic, s_iw = b[4], b[5], b[6]
        base = chunk_base(q)
        pltpu.async_copy(row_hbm.at[pl.ds(base, CCHUNK)], idxr, s_ir)
        pltpu.async_copy(col_hbm.at[pl.ds(base, CCHUNK)], idxc, s_ic)
        pltpu.async_copy(w_hbm.at[pl.ds(base, CCHUNK)], w_v, s_iw)

    def idx_wait(b):
        idxr, idxc, w_v = b[0], b[1], b[2]
        s_ir, s_ic, s_iw = b[4], b[5], b[6]
        pltpu.make_async_copy(row_hbm.at[pl.ds(0, CCHUNK)], idxr, s_ir).wait()
        pltpu.make_async_copy(col_hbm.at[pl.ds(0, CCHUNK)], idxc, s_ic).wait()
        pltpu.make_async_copy(w_hbm.at[pl.ds(0, CCHUNK)], w_v, s_iw).wait()

    def gather_start(b):
        idxr, rows_v, s_g = b[0], b[3], b[7]

        @pl.when(c == 0)
        def _():
            pltpu.async_copy(xa_hbm.at[idxr], rows_v, s_g)

        @pl.when(c == 1)
        def _():
            pltpu.async_copy(xb_hbm.at[idxr], rows_v, s_g)

    def gather_wait(b):
        idxr, rows_v, s_g = b[0], b[3], b[7]

        @pl.when(c == 0)
        def _():
            pltpu.make_async_copy(xa_hbm.at[idxr], rows_v, s_g).wait()

        @pl.when(c == 1)
        def _():
            pltpu.make_async_copy(xb_hbm.at[idxr], rows_v, s_g).wait()

    def scatter_start(b):
        idxc, rows_v, s_sc = b[1], b[3], b[8]
        pltpu.async_copy(rows_v, acc_sh.at[idxc], s_sc, add=True)

    def scatter_wait(b):
        idxc, rows_v, s_sc = b[1], b[3], b[8]
        pltpu.make_async_copy(rows_v, acc_sh.at[idxc], s_sc).wait()

    def multiply(b):
        w_v, rows_v = b[2], b[3]

        @pl.loop(0, CCHUNK, unroll=4)
        def _edge(e):
            ws = plsc.load_gather(w_v, [jnp.broadcast_to(e, (L,))])
            for k in range(H // L):
                sl = (e, pl.ds(k * L, L))
                rows_v[sl] = rows_v[sl] * ws

    def step(j, b, bn):
        gather_wait(b)

        @pl.when(j + 1 < nch)
        def _():
            @pl.when(j >= 1)
            def _():
                scatter_wait(bn)
            idx_start(j + 1, bn)

        multiply(b)
        scatter_start(b)

        @pl.when(j + 1 < nch)
        def _():
            idx_wait(bn)
            gather_start(bn)

    idx_start(0, b0)
    idx_wait(b0)
    gather_start(b0)

    @pl.loop(0, nch)
    def _chunk(j):
        @pl.when(j % 2 == 0)
        def _():
            step(j, b0, b1)

        @pl.when(j % 2 == 1)
        def _():
            step(j, b1, b0)

    scatter_wait(b0)
    scatter_wait(b1)

    plsc.subcore_barrier()

    rows0, rows1 = b0[3], b1[3]

    def normalize_rows(nrows, rows_v):
        for g in range(nrows // L):
            sl = pl.ds(g * L, L)
            inv_v[sl] = 1.0 / jnp.maximum(se0_v[sl] + se1_v[sl], 1e-16)

        @pl.loop(0, nrows, unroll=4)
        def _row(r):
            iv = plsc.load_gather(inv_v, [jnp.broadcast_to(r, (L,))])
            for k in range(H // L):
                sl = (r, pl.ds(k * L, L))
                rows_v[sl] = rows_v[sl] * iv

    @pl.loop(0, (NRCH - s + NS - 1) // NS)
    def _drain(t):
        r0 = (s + t * NS) * RCHUNK
        pltpu.sync_copy(se_hbm.at[pl.ds(r0, RCHUNK)], se0_v)
        pltpu.sync_copy(se_hbm.at[pl.ds(N + r0, RCHUNK)], se1_v)
        pltpu.sync_copy(acc_sh.at[pl.ds(r0, RCHUNK)], rows0.at[pl.ds(0, RCHUNK)])
        normalize_rows(RCHUNK, rows0)
        pltpu.sync_copy(rows0.at[pl.ds(0, RCHUNK)],
                        out_hbm.at[pl.ds(r0, RCHUNK), pl.ds(c * H, H)])

    @pl.when(s == 0)
    def _():
        r0 = NRCH * RCHUNK
        pltpu.sync_copy(se_hbm.at[pl.ds(r0, RTAIL)], se0_v.at[pl.ds(0, RTAIL)])
        pltpu.sync_copy(se_hbm.at[pl.ds(N + r0, RTAIL)],
                        se1_v.at[pl.ds(0, RTAIL)])
        pltpu.sync_copy(acc_sh.at[pl.ds(r0, RTAIL)], rows1.at[pl.ds(0, RTAIL)])
        normalize_rows(RTAIL, rows1)
        pltpu.sync_copy(rows1.at[pl.ds(0, RTAIL)],
                        out_hbm.at[pl.ds(r0, RTAIL), pl.ds(c * H, H)])


_scatter = functools.partial(
    pl.kernel,
    out_type=jax.ShapeDtypeStruct((N, D), jnp.float32),
    mesh=_mesh,
    compiler_params=_sc_params,
    scratch_types=(
        2 * [
            pltpu.VMEM((CCHUNK,), jnp.int32),
            pltpu.VMEM((CCHUNK,), jnp.int32),
            pltpu.VMEM((CCHUNK,), jnp.float32),
            pltpu.VMEM((CCHUNK, H), jnp.float32),
            pltpu.SemaphoreType.DMA,
            pltpu.SemaphoreType.DMA,
            pltpu.SemaphoreType.DMA,
            pltpu.SemaphoreType.DMA,
            pltpu.SemaphoreType.DMA,
        ]
        + [
            pltpu.VMEM_SHARED((N, H), jnp.float32),
            pltpu.VMEM((RCHUNK,), jnp.float32),
            pltpu.VMEM((RCHUNK,), jnp.float32),
            pltpu.VMEM((RCHUNK,), jnp.float32),
        ]
    ),
)(_scatter_body)


def kernel(x, edge_index, beta):
    x = x.astype(jnp.float32)
    row = edge_index[0].astype(jnp.int32)
    col = edge_index[1].astype(jnp.int32)
    beta16 = jnp.broadcast_to(beta.astype(jnp.float32), (L,))
    zeros_n = jnp.zeros((N,), jnp.float32)
    zacc = jnp.zeros((N, H), jnp.float32)

    xa, xb, rn2 = _prep(x)
    rn = rn2.reshape(N)
    w, sumexp = _score(x, row, col, rn, beta16, zeros_n)
    return _scatter(xa, xb, row, col, w, zacc, sumexp)
